# baseline (speedup 1.0000x reference)
import jax
import jax.numpy as jnp
from jax.experimental import pallas as pl
from jax.experimental.pallas import tpu as pltpu

TS_START = 26
N_TABLE = 126
N_REP = 20
N_TS = N_TABLE - TS_START
F_OUT = TS_START + N_TS * N_REP
DIM = 64
B_BLK = 8


def _add_kernel(x_ref, table_ref, o_ref, bias_ref):
    @pl.when(pl.program_id(0) == 0)
    def _build_bias():
        bias_ref[0:TS_START] = table_ref[0:TS_START]
        ts = table_ref[TS_START:N_TABLE]
        for r in range(N_REP):
            base = TS_START + r * N_TS
            bias_ref[base:base + N_TS] = ts

    o_ref[...] = x_ref[...] + bias_ref[...][None, :, :]


def kernel(X, table):
    B = X.shape[0]
    return pl.pallas_call(
        _add_kernel,
        grid=(B // B_BLK,),
        in_specs=[
            pl.BlockSpec((B_BLK, F_OUT, DIM), lambda i: (i, 0, 0)),
            pl.BlockSpec((N_TABLE, DIM), lambda i: (0, 0)),
        ],
        out_specs=pl.BlockSpec((B_BLK, F_OUT, DIM), lambda i: (i, 0, 0)),
        out_shape=jax.ShapeDtypeStruct((B, F_OUT, DIM), X.dtype),
        scratch_shapes=[pltpu.VMEM((F_OUT, DIM), X.dtype)],
    )(X, table)


# speedup vs baseline: 1.7652x; 1.7652x over previous
import jax
import jax.numpy as jnp
from jax.experimental import pallas as pl
from jax.experimental.pallas import tpu as pltpu

TS_START = 26
N_TABLE = 126
N_REP = 20
N_TS = N_TABLE - TS_START
F_OUT = TS_START + N_TS * N_REP
DIM = 64
W = F_OUT * DIM
B_BLK = 8


def _bias_kernel(table_ref, full_ref):
    full_ref[0:TS_START] = table_ref[0:TS_START]
    ts = table_ref[TS_START:N_TABLE]
    for r in range(N_REP):
        base = TS_START + r * N_TS
        full_ref[base:base + N_TS] = ts


def _add_kernel(x_ref, b_ref, o_ref):
    o_ref[...] = x_ref[...] + b_ref[...]


def kernel(X, table):
    B = X.shape[0]
    full2d = pl.pallas_call(
        _bias_kernel,
        out_shape=jax.ShapeDtypeStruct((F_OUT, DIM), table.dtype),
    )(table)
    bias_row = full2d.reshape(1, W)
    X2 = X.reshape(B, W)
    out = pl.pallas_call(
        _add_kernel,
        grid=(B // B_BLK,),
        in_specs=[
            pl.BlockSpec((B_BLK, W), lambda i: (i, 0)),
            pl.BlockSpec((1, W), lambda i: (0, 0)),
        ],
        out_specs=pl.BlockSpec((B_BLK, W), lambda i: (i, 0)),
        out_shape=jax.ShapeDtypeStruct((B, W), X.dtype),
    )(X2, bias_row)
    return out.reshape(B, F_OUT, DIM)
